# Initial kernel scaffold; baseline (speedup 1.0000x reference)
import jax
import jax.numpy as jnp
from jax.experimental import pallas as pl


def kernel(x, edge_index, W, b, ln_gamma, ln_beta):
    raise NotImplementedError("write your pallas kernel here")


# speedup vs baseline: 13.7179x; 13.7179x over previous
import functools

import jax
import jax.numpy as jnp
from jax import lax
from jax.experimental import pallas as pl
from jax.experimental.pallas import tpu as pltpu
from jax.experimental.pallas import tpu_sc as plsc

N = 10000
D = 128
E = 320000
NC = 2
NS = 16
NW = NC * NS
C = 96
NCHUNK = 106
EPP = C * NCHUNK
EPAD = NW * EPP
SINK = N
ACCR = 10008
NPAD = 10240
RPW = N // NS
ZR = 125

_vmesh = functools.partial(
    plsc.VectorSubcoreMesh, core_axis_name="c", subcore_axis_name="s")


def _deg_body(dst_hbm, zeros1_hbm, degp_hbm, acc, dstv, ones, sem):
  c = lax.axis_index("c")
  s = lax.axis_index("s")
  w = c * NS + s
  pltpu.sync_copy(zeros1_hbm.at[pl.ds(s * (NPAD // NS), NPAD // NS)],
                  acc.at[pl.ds(s * (NPAD // NS), NPAD // NS)])
  @pl.loop(0, 8)
  def _(i):
    ones[pl.ds(i * 16, 16)] = jnp.ones((16,), jnp.float32)
  pltpu.async_copy(dst_hbm.at[w], dstv, sem).wait()
  plsc.subcore_barrier()

  @pl.loop(0, NCHUNK)
  def _(j):
    pltpu.sync_copy(ones.at[pl.ds(0, C)], acc.at[dstv.at[j]], add=True)

  plsc.subcore_barrier()
  @pl.when(s == 0)
  def _():
    pltpu.sync_copy(acc, degp_hbm.at[c])


def _deg_partials(dst3, zeros1):
  return pl.kernel(
      _deg_body,
      out_type=jax.ShapeDtypeStruct((NC, NPAD), jnp.float32),
      mesh=_vmesh(),
      scratch_types=[
          pltpu.VMEM_SHARED((NPAD,), jnp.float32),
          pltpu.VMEM((NCHUNK, C), jnp.int32),
          pltpu.VMEM((128,), jnp.float32),
          pltpu.SemaphoreType.DMA,
      ],
  )(dst3, zeros1)


def _scale_body(x_ref, w_ref, d0_ref, d1_ref, scaled_ref, dis_ref):
  deg = d0_ref[...] + d1_ref[...] + 1.0
  dis = 1.0 / jnp.sqrt(deg)
  h = jnp.dot(x_ref[...], w_ref[...], preferred_element_type=jnp.float32)
  scaled_ref[...] = h * dis
  dis_ref[...] = dis


def _matmul_scale(x, W, deg0, deg1):
  B = 400
  grid = (N // B,)
  return pl.pallas_call(
      _scale_body,
      grid=grid,
      in_specs=[
          pl.BlockSpec((B, D), lambda i: (i, 0)),
          pl.BlockSpec((D, D), lambda i: (0, 0)),
          pl.BlockSpec((B, 1), lambda i: (i, 0)),
          pl.BlockSpec((B, 1), lambda i: (i, 0)),
      ],
      out_specs=[
          pl.BlockSpec((B, D), lambda i: (i, 0)),
          pl.BlockSpec((B, 1), lambda i: (i, 0)),
      ],
      out_shape=[
          jax.ShapeDtypeStruct((N, D), jnp.float32),
          jax.ShapeDtypeStruct((N, 1), jnp.float32),
      ],
  )(x, W, deg0, deg1)


def _agg_body(scaled_hbm, src_hbm, dst_hbm, zeros2_hbm, part_hbm,
              acc, srcv, dstv, rows0, rows1, sem0, sem1, isem):
  c = lax.axis_index("c")
  s = lax.axis_index("s")
  w = c * NS + s
  @pl.loop(0, RPW // ZR)
  def _(i):
    pltpu.sync_copy(zeros2_hbm, acc.at[pl.ds(s * RPW + i * ZR, ZR)])
  pltpu.async_copy(src_hbm.at[pl.ds(w * EPP, EPP)], srcv, isem).wait()
  pltpu.async_copy(dst_hbm.at[w], dstv, isem).wait()
  plsc.subcore_barrier()

  pltpu.async_copy(scaled_hbm.at[srcv.at[pl.ds(0, C)]], rows0, sem0)

  @pl.loop(0, NCHUNK, step=2)
  def _(j):
    pltpu.make_async_copy(
        scaled_hbm.at[srcv.at[pl.ds(j * C, C)]], rows0, sem0).wait()
    pltpu.async_copy(
        scaled_hbm.at[srcv.at[pl.ds((j + 1) * C, C)]], rows1, sem1)
    pltpu.sync_copy(rows0, acc.at[dstv.at[j]], add=True)
    pltpu.make_async_copy(
        scaled_hbm.at[srcv.at[pl.ds((j + 1) * C, C)]], rows1, sem1).wait()
    @pl.when(j + 2 < NCHUNK)
    def _():
      pltpu.async_copy(
          scaled_hbm.at[srcv.at[pl.ds((j + 2) * C, C)]], rows0, sem0)
    pltpu.sync_copy(rows1, acc.at[dstv.at[j + 1]], add=True)

  plsc.subcore_barrier()
  pltpu.sync_copy(acc.at[pl.ds(s * RPW, RPW)], part_hbm.at[w])


def _edge_aggregate(scaled, src_flat, dst3, zeros2):
  return pl.kernel(
      _agg_body,
      out_type=jax.ShapeDtypeStruct((NW, RPW, D), jnp.float32),
      mesh=_vmesh(),
      scratch_types=[
          pltpu.VMEM_SHARED((ACCR, D), jnp.float32),
          pltpu.VMEM((EPP,), jnp.int32),
          pltpu.VMEM((NCHUNK, C), jnp.int32),
          pltpu.VMEM((C, D), jnp.float32),
          pltpu.VMEM((C, D), jnp.float32),
          pltpu.SemaphoreType.DMA,
          pltpu.SemaphoreType.DMA,
          pltpu.SemaphoreType.DMA,
      ],
  )(scaled, src_flat, dst3, zeros2)


def _ln_body(p0_ref, p1_ref, sc_ref, dis_ref, x_ref, b_ref, g_ref, bt_ref,
             out_ref):
  agg = (p0_ref[...] + p1_ref[...] + sc_ref[...]) * dis_ref[...] + b_ref[...]
  mu = jnp.mean(agg, axis=-1, keepdims=True)
  zc = agg - mu
  var = jnp.mean(zc * zc, axis=-1, keepdims=True)
  ln = zc / jnp.sqrt(var + 1e-5) * g_ref[...] + bt_ref[...]
  out_ref[...] = jnp.maximum(ln, 0.0) + x_ref[...]


def _ln_residual(p0, p1, scaled, dis, x, b, g, bt):
  B = 400
  grid = (N // B,)
  row = lambda i: (i, 0)
  return pl.pallas_call(
      _ln_body,
      grid=grid,
      in_specs=[
          pl.BlockSpec((B, D), row),
          pl.BlockSpec((B, D), row),
          pl.BlockSpec((B, D), row),
          pl.BlockSpec((B, 1), row),
          pl.BlockSpec((B, D), row),
          pl.BlockSpec((1, D), lambda i: (0, 0)),
          pl.BlockSpec((1, D), lambda i: (0, 0)),
          pl.BlockSpec((1, D), lambda i: (0, 0)),
      ],
      out_specs=pl.BlockSpec((B, D), row),
      out_shape=jax.ShapeDtypeStruct((N, D), jnp.float32),
  )(p0, p1, scaled, dis, x, b, g, bt)


def kernel(x, edge_index, W, b, ln_gamma, ln_beta):
  ei = edge_index.astype(jnp.int32)
  npad = EPAD - E
  src_flat = jnp.concatenate([ei[0], jnp.zeros((npad,), jnp.int32)])
  dst_flat = jnp.concatenate([ei[1], jnp.full((npad,), SINK, jnp.int32)])
  dst3 = dst_flat.reshape(NW, NCHUNK, C)
  zeros1 = jnp.zeros((NPAD,), jnp.float32)
  zeros2 = jnp.zeros((ZR, D), jnp.float32)

  degp = _deg_partials(dst3, zeros1)
  deg0 = degp[0, :N].reshape(N, 1)
  deg1 = degp[1, :N].reshape(N, 1)

  scaled, dis = _matmul_scale(x, W, deg0, deg1)

  parts = _edge_aggregate(scaled, src_flat, dst3, zeros2)
  p = parts.reshape(NC, N, D)

  return _ln_residual(p[0], p[1], scaled, dis, x, b.reshape(1, D),
                      ln_gamma.reshape(1, D), ln_beta.reshape(1, D))
